# baseline (speedup 1.0000x reference)
import functools

import jax
import jax.numpy as jnp
import numpy as _np
from jax import lax
from jax.experimental import pallas as pl
from jax.experimental.pallas import tpu as pltpu
from jax.experimental.pallas import tpu_sc as plsc

_LANES = 16
_NC = 2
_NS = 16
_NW = _NC * _NS

_LP = 208
_NBUF = 4

_UNPACK_PERM = _np.concatenate(
    [_np.concatenate([_np.arange(0, 32, 2), _np.arange(1, 32, 2)]) + 32 * c
     for c in range(2)])


def _embbag_sc(idx_pad, table_pk):
    B, LP = idx_pad.shape
    E = table_pk.shape[1] * 2
    SPT = B // _NW
    NCH = E // 32

    mesh = plsc.VectorSubcoreMesh(
        core_axis_name="c", subcore_axis_name="s",
        num_cores=_NC, num_subcores=_NS)

    @functools.partial(
        pl.kernel,
        mesh=mesh,
        out_type=jax.ShapeDtypeStruct((B, E), jnp.float32),
        scratch_types=[
            pltpu.VMEM((SPT, LP), jnp.int32),
            pltpu.VMEM((_NBUF, LP, E // 2), jnp.int32),
            pltpu.VMEM((SPT, E), jnp.float32),
        ] + [pltpu.SemaphoreType.DMA] * _NBUF,
        compiler_params=pltpu.CompilerParams(
            use_tc_tiling_on_sc=False, needs_layout_passes=False),
    )
    def body(idx_hbm, table_hbm, pooled_hbm, idx_v, rows_v, pool_v, *sems):
        wid = lax.axis_index("s") * _NC + lax.axis_index("c")
        base = wid * SPT
        pltpu.sync_copy(idx_hbm.at[pl.ds(base, SPT)], idx_v)

        def fire(s, b):
            pltpu.async_copy(
                table_hbm.at[idx_v.at[s]], rows_v.at[b], sems[b])

        def wait(b):
            pltpu.make_async_copy(
                table_hbm.at[pl.ds(0, LP)], rows_v.at[b], sems[b]).wait()

        for b in range(_NBUF):
            fire(b, b)

        fzero = jnp.zeros((_LANES,), jnp.float32)
        ione = jnp.ones((_LANES,), jnp.int32)
        izero = jnp.zeros((_LANES,), jnp.int32)

        def group(g, carry):
            for b in range(_NBUF):
                s = g * _NBUF + b
                wait(b)

                def jbody(j, accs):
                    out = list(accs)
                    for c in range(NCH):
                        x = rows_v[b, j, pl.ds(c * _LANES, _LANES)]
                        xb = plsc.bitcast(x, jnp.bfloat16)
                        lo, hi = plsc.unpack(
                            xb, format=plsc.PackFormat.INTERLEAVED,
                            preferred_element_type=jnp.float32)
                        out[2 * c] = out[2 * c] + lo
                        out[2 * c + 1] = out[2 * c + 1] + hi
                    return tuple(out)

                accs = lax.fori_loop(0, LP, jbody, (fzero,) * (2 * NCH))

                ns = s + _NBUF

                @pl.when(ns < SPT)
                def _():
                    fire(ns, b)

                def cbody(k, cv):
                    iv = idx_v[s, pl.ds(k * _LANES, _LANES)]
                    return cv + jnp.where(iv != 0, ione, izero)

                cv = lax.fori_loop(0, LP // _LANES, cbody, izero)
                cnt = jnp.maximum(jnp.sum(cv), 1)
                cntf = jnp.full((_LANES,), cnt.astype(jnp.float32))
                for c in range(2 * NCH):
                    pool_v[s, pl.ds(c * _LANES, _LANES)] = accs[c] / cntf
            return carry

        lax.fori_loop(0, SPT // _NBUF, group, 0)
        pltpu.sync_copy(pool_v, pooled_hbm.at[pl.ds(base, SPT)])

    return body(idx_pad, table_pk)


def _mlp_tc(pooled, W1, b1, W2, b2):
    B, E = pooled.shape
    H = W1.shape[1]
    C = W2.shape[1]
    BT = 512

    def body(x_ref, w1_ref, b1_ref, w2_ref, b2_ref, o_ref):
        x = x_ref[...]
        h = jnp.dot(x, w1_ref[...], preferred_element_type=jnp.float32)
        h = jnp.maximum(h + b1_ref[...], 0.0)
        logits = jnp.dot(h, w2_ref[...], preferred_element_type=jnp.float32)
        logits = logits + b2_ref[...]
        m = jnp.max(logits, axis=1, keepdims=True)
        e = jnp.exp(logits - m)
        o_ref[...] = e / jnp.sum(e, axis=1, keepdims=True)

    return pl.pallas_call(
        body,
        grid=(B // BT,),
        in_specs=[
            pl.BlockSpec((BT, E), lambda i: (i, 0)),
            pl.BlockSpec((E, H), lambda i: (0, 0)),
            pl.BlockSpec((1, H), lambda i: (0, 0)),
            pl.BlockSpec((H, C), lambda i: (0, 0)),
            pl.BlockSpec((1, C), lambda i: (0, 0)),
        ],
        out_specs=pl.BlockSpec((BT, C), lambda i: (i, 0)),
        out_shape=jax.ShapeDtypeStruct((B, C), jnp.float32),
    )(pooled, W1, b1.reshape(1, H), W2, b2.reshape(1, C))


def kernel(batch_inputs, batch_lengths, emb_table, W1, b1, W2, b2):
    B, L = batch_inputs.shape
    idx_pad = jnp.pad(batch_inputs, ((0, 0), (0, _LP - L)))
    V, E = emb_table.shape
    table_pk = lax.bitcast_convert_type(
        emb_table.astype(jnp.bfloat16).reshape(V, E // 2, 2), jnp.int32)
    pooled = _embbag_sc(idx_pad, table_pk)
    W1p = W1[_UNPACK_PERM, :]
    return _mlp_tc(pooled, W1p, b1, W2, b2)


# speedup vs baseline: 1.5483x; 1.5483x over previous
import functools

import jax
import jax.numpy as jnp
from jax import lax
from jax.experimental import pallas as pl
from jax.experimental.pallas import tpu as pltpu
from jax.experimental.pallas import tpu_sc as plsc

_LANES = 16
_NC = 2
_NS = 16
_NW = _NC * _NS

_LP = 208
_NBUF = 4


def _embbag_sc(idx_pad, table):
    B, LP = idx_pad.shape
    E = table.shape[1]
    SPT = B // _NW
    NCH = E // _LANES

    mesh = plsc.VectorSubcoreMesh(
        core_axis_name="c", subcore_axis_name="s",
        num_cores=_NC, num_subcores=_NS)

    @functools.partial(
        pl.kernel,
        mesh=mesh,
        out_type=jax.ShapeDtypeStruct((B, E), jnp.float32),
        scratch_types=[
            pltpu.VMEM((SPT, LP), jnp.int32),
            pltpu.VMEM((_NBUF, LP, E), jnp.float32),
            pltpu.VMEM((SPT, E), jnp.float32),
        ] + [pltpu.SemaphoreType.DMA] * _NBUF,
        compiler_params=pltpu.CompilerParams(
            use_tc_tiling_on_sc=False, needs_layout_passes=False),
    )
    def body(idx_hbm, table_hbm, pooled_hbm, idx_v, rows_v, pool_v, *sems):
        wid = lax.axis_index("s") * _NC + lax.axis_index("c")
        base = wid * SPT
        pltpu.sync_copy(idx_hbm.at[pl.ds(base, SPT)], idx_v)

        def fire(s, b):
            pltpu.async_copy(
                table_hbm.at[idx_v.at[s]], rows_v.at[b], sems[b])

        def wait(b):
            pltpu.make_async_copy(
                table_hbm.at[pl.ds(0, LP)], rows_v.at[b], sems[b]).wait()

        for b in range(_NBUF):
            fire(b, b)

        fzero = jnp.zeros((_LANES,), jnp.float32)
        ione = jnp.ones((_LANES,), jnp.int32)
        izero = jnp.zeros((_LANES,), jnp.int32)

        def group(g, carry):
            for b in range(_NBUF):
                s = g * _NBUF + b
                wait(b)

                def jbody(j, accs):
                    out = list(accs)
                    for c in range(NCH):
                        out[c] = out[c] + rows_v[b, j, pl.ds(c * _LANES, _LANES)]
                    return tuple(out)

                accs = lax.fori_loop(0, LP, jbody, (fzero,) * NCH)

                ns = s + _NBUF

                @pl.when(ns < SPT)
                def _():
                    fire(ns, b)

                def cbody(k, cv):
                    iv = idx_v[s, pl.ds(k * _LANES, _LANES)]
                    return cv + jnp.where(iv != 0, ione, izero)

                cv = lax.fori_loop(0, LP // _LANES, cbody, izero)
                cnt = jnp.maximum(jnp.sum(cv), 1)
                cntf = jnp.full((_LANES,), cnt.astype(jnp.float32))
                for c in range(NCH):
                    pool_v[s, pl.ds(c * _LANES, _LANES)] = accs[c] / cntf
            return carry

        lax.fori_loop(0, SPT // _NBUF, group, 0)
        pltpu.sync_copy(pool_v, pooled_hbm.at[pl.ds(base, SPT)])

    return body(idx_pad, table)


def _mlp_tc(pooled, W1, b1, W2, b2):
    B, E = pooled.shape
    H = W1.shape[1]
    C = W2.shape[1]
    BT = 512

    def body(x_ref, w1_ref, b1_ref, w2_ref, b2_ref, o_ref):
        x = x_ref[...]
        h = jnp.dot(x, w1_ref[...], preferred_element_type=jnp.float32)
        h = jnp.maximum(h + b1_ref[...], 0.0)
        logits = jnp.dot(h, w2_ref[...], preferred_element_type=jnp.float32)
        logits = logits + b2_ref[...]
        m = jnp.max(logits, axis=1, keepdims=True)
        e = jnp.exp(logits - m)
        o_ref[...] = e / jnp.sum(e, axis=1, keepdims=True)

    return pl.pallas_call(
        body,
        grid=(B // BT,),
        in_specs=[
            pl.BlockSpec((BT, E), lambda i: (i, 0)),
            pl.BlockSpec((E, H), lambda i: (0, 0)),
            pl.BlockSpec((1, H), lambda i: (0, 0)),
            pl.BlockSpec((H, C), lambda i: (0, 0)),
            pl.BlockSpec((1, C), lambda i: (0, 0)),
        ],
        out_specs=pl.BlockSpec((BT, C), lambda i: (i, 0)),
        out_shape=jax.ShapeDtypeStruct((B, C), jnp.float32),
    )(pooled, W1, b1.reshape(1, H), W2, b2.reshape(1, C))


def kernel(batch_inputs, batch_lengths, emb_table, W1, b1, W2, b2):
    B, L = batch_inputs.shape
    idx_pad = jnp.pad(batch_inputs, ((0, 0), (0, _LP - L)))
    pooled = _embbag_sc(idx_pad, emb_table)
    return _mlp_tc(pooled, W1, b1, W2, b2)


# speedup vs baseline: 1.5562x; 1.0051x over previous
import functools

import jax
import jax.numpy as jnp
from jax import lax
from jax.experimental import pallas as pl
from jax.experimental.pallas import tpu as pltpu
from jax.experimental.pallas import tpu_sc as plsc

_LANES = 16
_NC = 2
_NS = 16
_NW = _NC * _NS

_LP = 208
_G = 2
_NBUF = 2


def _embbag_sc(idx_pad, table):
    NG_ALL, GLP = idx_pad.shape
    B = NG_ALL * _G
    E = table.shape[1]
    SPT = B // _NW
    NGR = SPT // _G
    NCH = E // _LANES

    mesh = plsc.VectorSubcoreMesh(
        core_axis_name="c", subcore_axis_name="s",
        num_cores=_NC, num_subcores=_NS)

    @functools.partial(
        pl.kernel,
        mesh=mesh,
        out_type=jax.ShapeDtypeStruct((B, E), jnp.float32),
        scratch_types=[
            pltpu.VMEM((NGR, GLP), jnp.int32),
            pltpu.VMEM((_NBUF, GLP, E), jnp.float32),
            pltpu.VMEM((SPT, E), jnp.float32),
        ] + [pltpu.SemaphoreType.DMA] * _NBUF,
        compiler_params=pltpu.CompilerParams(
            use_tc_tiling_on_sc=False, needs_layout_passes=False),
    )
    def body(idx_hbm, table_hbm, pooled_hbm, idx_v, rows_v, pool_v, *sems):
        wid = lax.axis_index("s") * _NC + lax.axis_index("c")
        base = wid * NGR
        pltpu.sync_copy(idx_hbm.at[pl.ds(base, NGR)], idx_v)

        def fire(g, b):
            pltpu.async_copy(
                table_hbm.at[idx_v.at[g]], rows_v.at[b], sems[b])

        def wait(b):
            pltpu.make_async_copy(
                table_hbm.at[pl.ds(0, GLP)], rows_v.at[b], sems[b]).wait()

        for b in range(_NBUF):
            fire(b, b)

        fzero = jnp.zeros((_LANES,), jnp.float32)
        ione = jnp.ones((_LANES,), jnp.int32)
        izero = jnp.zeros((_LANES,), jnp.int32)

        def round_(r, carry):
            for b in range(_NBUF):
                g = r * _NBUF + b
                wait(b)

                for i in range(_G):
                    def jbody(j, accs):
                        out = list(accs)
                        for c in range(NCH):
                            out[c] = out[c] + rows_v[
                                b, i * _LP + j, pl.ds(c * _LANES, _LANES)]
                        return tuple(out)

                    accs = lax.fori_loop(0, _LP, jbody, (fzero,) * NCH)

                    def cbody(k, cv):
                        iv = idx_v[
                            g, pl.ds(i * _LP + k * _LANES, _LANES)]
                        return cv + jnp.where(iv != 0, ione, izero)

                    cv = lax.fori_loop(0, _LP // _LANES, cbody, izero)
                    cnt = jnp.maximum(jnp.sum(cv), 1)
                    cntf = jnp.full((_LANES,), cnt.astype(jnp.float32))
                    s = g * _G + i
                    for c in range(NCH):
                        pool_v[s, pl.ds(c * _LANES, _LANES)] = accs[c] / cntf

                ng = g + _NBUF

                @pl.when(ng < NGR)
                def _():
                    fire(ng, b)
            return carry

        lax.fori_loop(0, NGR // _NBUF, round_, 0)
        pltpu.sync_copy(pool_v, pooled_hbm.at[pl.ds(wid * SPT, SPT)])

    return body(idx_pad, table)


def _mlp_tc(pooled, W1, b1, W2, b2):
    B, E = pooled.shape
    H = W1.shape[1]
    C = W2.shape[1]
    BT = 512

    def body(x_ref, w1_ref, b1_ref, w2_ref, b2_ref, o_ref):
        x = x_ref[...]
        h = jnp.dot(x, w1_ref[...], preferred_element_type=jnp.float32)
        h = jnp.maximum(h + b1_ref[...], 0.0)
        logits = jnp.dot(h, w2_ref[...], preferred_element_type=jnp.float32)
        logits = logits + b2_ref[...]
        m = jnp.max(logits, axis=1, keepdims=True)
        e = jnp.exp(logits - m)
        o_ref[...] = e / jnp.sum(e, axis=1, keepdims=True)

    return pl.pallas_call(
        body,
        grid=(B // BT,),
        in_specs=[
            pl.BlockSpec((BT, E), lambda i: (i, 0)),
            pl.BlockSpec((E, H), lambda i: (0, 0)),
            pl.BlockSpec((1, H), lambda i: (0, 0)),
            pl.BlockSpec((H, C), lambda i: (0, 0)),
            pl.BlockSpec((1, C), lambda i: (0, 0)),
        ],
        out_specs=pl.BlockSpec((BT, C), lambda i: (i, 0)),
        out_shape=jax.ShapeDtypeStruct((B, C), jnp.float32),
    )(pooled, W1, b1.reshape(1, H), W2, b2.reshape(1, C))


def kernel(batch_inputs, batch_lengths, emb_table, W1, b1, W2, b2):
    B, L = batch_inputs.shape
    idx_pad = jnp.pad(batch_inputs, ((0, 0), (0, _LP - L)))
    idx_pad = idx_pad.reshape(B // _G, _G * _LP)
    pooled = _embbag_sc(idx_pad, emb_table)
    return _mlp_tc(pooled, W1, b1, W2, b2)
